# baseline (speedup 1.0000x reference)
import functools

import jax
import jax.numpy as jnp
from jax import lax
from jax.experimental import pallas as pl
from jax.experimental.pallas import tpu as pltpu
from jax.experimental.pallas import tpu_sc as plsc

_NUM_CORES = 2
_NUM_SUBCORES = 16
_NUM_WORKERS = _NUM_CORES * _NUM_SUBCORES

_CHUNK = 256
_NBUF = 2

_RETILE_W = 4096


@functools.lru_cache(maxsize=None)
def _make_retile(vocab: int, d: int):
  grid = -(-vocab // _RETILE_W)

  def retile_body(tT_ref, out_ref):
    x = tT_ref[...]
    xt = jnp.transpose(x)
    y = jnp.concatenate([xt[:_RETILE_W // 2], xt[_RETILE_W // 2:]], axis=1)
    out_ref[...] = jnp.reshape(y, (_RETILE_W * d,))

  return pl.pallas_call(
      retile_body,
      grid=(grid,),
      in_specs=[pl.BlockSpec((d, _RETILE_W), lambda i: (0, i))],
      out_specs=pl.BlockSpec((_RETILE_W * d,), lambda i: (i,)),
      out_shape=jax.ShapeDtypeStruct((grid * _RETILE_W * d,), jnp.float32),
  )


@functools.lru_cache(maxsize=None)
def _make_gather(n: int, table_rows: int, d: int, s_len: int, b_len: int):
  n_per_w = n // _NUM_WORKERS
  n_chunks = n_per_w // _CHUNK
  chunks_per_s = b_len // _CHUNK
  n_bt = _CHUNK // 128
  assert n == n_per_w * _NUM_WORKERS
  assert n_per_w == n_chunks * _CHUNK
  assert n_chunks % _NBUF == 0 and b_len % _CHUNK == 0
  n_groups = n_chunks // _NBUF
  mesh = plsc.VectorSubcoreMesh(
      core_axis_name="c", subcore_axis_name="s",
      num_cores=_NUM_CORES, num_subcores=_NUM_SUBCORES)

  @functools.partial(
      pl.kernel,
      out_type=jax.ShapeDtypeStruct(
          (s_len, d // 8, b_len // 128, 8, 128), jnp.float32),
      mesh=mesh,
      compiler_params=pltpu.CompilerParams(use_tc_tiling_on_sc=False,
                                           needs_layout_passes=False),
      scratch_types=[
          pltpu.VMEM((n_per_w,), jnp.int32),
          [pltpu.VMEM((_CHUNK, d), jnp.float32) for _ in range(_NBUF)],
          [pltpu.VMEM((d // 8, n_bt, 8, 128), jnp.float32)
           for _ in range(_NBUF)],
          [pltpu.SemaphoreType.DMA for _ in range(_NBUF)],
          [pltpu.SemaphoreType.DMA for _ in range(_NBUF)],
      ],
  )
  def gather_kernel(idx_hbm, table_hbm, out_hbm, idx_all, rows, tbuf,
                    gsem, osem):
    wid = lax.axis_index("s") * _NUM_CORES + lax.axis_index("c")
    base = wid * n_per_w
    chunk0 = wid * n_chunks
    pltpu.sync_copy(idx_hbm.at[pl.ds(base, n_per_w)], idx_all)
    lanes = lax.iota(jnp.int32, 16)

    def fire_gather(i, b):
      pltpu.async_copy(
          table_hbm.at[idx_all.at[pl.ds(i * _CHUNK, _CHUNK)]],
          rows[b], gsem[b])

    def wait_gather(b):
      pltpu.make_async_copy(table_hbm.at[idx_all.at[pl.ds(0, _CHUNK)]],
                            rows[b], gsem[b]).wait()

    def fire_out(i, b):
      c = chunk0 + i
      s_ix = c // chunks_per_s
      bt0 = (c % chunks_per_s) * n_bt
      pltpu.async_copy(
          tbuf[b],
          out_hbm.at[s_ix, pl.ds(0, d // 8), pl.ds(bt0, n_bt)],
          osem[b])

    def wait_out(b):
      pltpu.make_async_copy(
          tbuf[b],
          out_hbm.at[0, pl.ds(0, d // 8), pl.ds(0, n_bt)],
          osem[b]).wait()

    def transpose_chunk(b):
      def body(q, carry):
        dt = q // (n_bt * 64)
        bt = (q // 64) & (n_bt - 1)
        ds = (q >> 3) & 7
        ln0 = (q & 7) << 4
        rowv = (bt * 128 + ln0) + lanes
        colv = jnp.full((16,), dt * 8 + ds, jnp.int32)
        v = plsc.load_gather(rows[b], [rowv, colv])
        tbuf[b][dt, bt, ds, pl.ds(ln0, 16)] = v
        return carry
      lax.fori_loop(0, (d // 8) * n_bt * 64, body, 0)

    for b in range(_NBUF):
      fire_gather(b, b)

    def group_body(g, carry):
      for b in range(_NBUF):
        i = g * _NBUF + b
        wait_gather(b)
        transpose_chunk(b)
        fire_gather(i + _NBUF, b)
        fire_out(i, b)
      for b in range(_NBUF):
        wait_out(b)
      return carry

    lax.fori_loop(0, n_groups - 1, group_body, 0)

    for b in range(_NBUF):
      i = (n_groups - 1) * _NBUF + b
      wait_gather(b)
      transpose_chunk(b)
      fire_out(i, b)
    for b in range(_NBUF):
      wait_out(b)

  return gather_kernel


def kernel(input, table):
  b, s, _ = input.shape
  vocab, d = table.shape
  n = b * s
  grid = -(-vocab // _RETILE_W)
  table_rows = jnp.reshape(_make_retile(vocab, d)(table.T),
                           (grid * _RETILE_W, d))
  v = jnp.reshape(jnp.transpose(input, (1, 2, 0)), (n,)).astype(jnp.int32)
  q = v & (_RETILE_W - 1)
  gidx = ((v >> 12) << 12) + ((q & (_RETILE_W // 2 - 1)) << 1) + (q >> 11)
  out5 = _make_gather(n, grid * _RETILE_W, d, s, b)(gidx, table_rows)
  return jnp.reshape(jnp.transpose(out5, (2, 4, 0, 1, 3)), (b, s, d))


# speedup vs baseline: 1.0317x; 1.0317x over previous
import functools

import jax
import jax.numpy as jnp
from jax import lax
from jax.experimental import pallas as pl
from jax.experimental.pallas import tpu as pltpu
from jax.experimental.pallas import tpu_sc as plsc

_NUM_CORES = 2
_NUM_SUBCORES = 16
_NUM_WORKERS = _NUM_CORES * _NUM_SUBCORES

_CHUNK = 256
_NBUF = 2

_RETILE_W = 4096


@functools.lru_cache(maxsize=None)
def _make_retile(vocab: int, d: int):
  grid = -(-vocab // _RETILE_W)

  def retile_body(tT_ref, out_ref):
    x = tT_ref[...]
    xt = jnp.transpose(x)
    y = jnp.concatenate([xt[:_RETILE_W // 2], xt[_RETILE_W // 2:]], axis=1)
    out_ref[...] = jnp.reshape(y, (_RETILE_W * d,))

  return pl.pallas_call(
      retile_body,
      grid=(grid,),
      in_specs=[pl.BlockSpec((d, _RETILE_W), lambda i: (0, i))],
      out_specs=pl.BlockSpec((_RETILE_W * d,), lambda i: (i,)),
      out_shape=jax.ShapeDtypeStruct((grid * _RETILE_W * d,), jnp.float32),
  )


@functools.lru_cache(maxsize=None)
def _make_gather(n: int, table_rows: int, d: int, s_len: int, b_len: int):
  n_per_w = n // _NUM_WORKERS
  n_chunks = n_per_w // _CHUNK
  chunks_per_s = b_len // _CHUNK
  n_bt = _CHUNK // 128
  assert n == n_per_w * _NUM_WORKERS
  assert n_per_w == n_chunks * _CHUNK
  assert n_chunks % _NBUF == 0 and b_len % _CHUNK == 0
  n_groups = n_chunks // _NBUF
  mesh = plsc.VectorSubcoreMesh(
      core_axis_name="c", subcore_axis_name="s",
      num_cores=_NUM_CORES, num_subcores=_NUM_SUBCORES)

  @functools.partial(
      pl.kernel,
      out_type=jax.ShapeDtypeStruct(
          (s_len, d // 8, b_len // 128, 8, 128), jnp.float32),
      mesh=mesh,
      compiler_params=pltpu.CompilerParams(use_tc_tiling_on_sc=False,
                                           needs_layout_passes=False),
      scratch_types=[
          pltpu.VMEM((n_per_w,), jnp.int32),
          [pltpu.VMEM((_CHUNK, d), jnp.float32) for _ in range(_NBUF)],
          [pltpu.VMEM((d // 8, n_bt, 8, 128), jnp.float32)
           for _ in range(_NBUF)],
          [pltpu.SemaphoreType.DMA for _ in range(_NBUF)],
          [pltpu.SemaphoreType.DMA for _ in range(_NBUF)],
      ],
  )
  def gather_kernel(idx_hbm, table_hbm, out_hbm, idx_all, rows, tbuf,
                    gsem, osem):
    wid = lax.axis_index("s") * _NUM_CORES + lax.axis_index("c")
    base = wid * n_per_w
    chunk0 = wid * n_chunks
    pltpu.sync_copy(idx_hbm.at[pl.ds(base, n_per_w)], idx_all)
    lanes = lax.iota(jnp.int32, 16)

    def fire_gather(i, b):
      pltpu.async_copy(
          table_hbm.at[idx_all.at[pl.ds(i * _CHUNK, _CHUNK)]],
          rows[b], gsem[b])

    def wait_gather(b):
      pltpu.make_async_copy(table_hbm.at[idx_all.at[pl.ds(0, _CHUNK)]],
                            rows[b], gsem[b]).wait()

    def fire_out(i, b):
      c = chunk0 + i
      s_ix = c // chunks_per_s
      bt0 = (c % chunks_per_s) * n_bt
      pltpu.async_copy(
          tbuf[b],
          out_hbm.at[s_ix, pl.ds(0, d // 8), pl.ds(bt0, n_bt)],
          osem[b])

    def wait_out(b):
      pltpu.make_async_copy(
          tbuf[b],
          out_hbm.at[0, pl.ds(0, d // 8), pl.ds(0, n_bt)],
          osem[b]).wait()

    def transpose_chunk(b):
      def body(o, carry):
        dt = o // n_bt
        bt = o % n_bt
        row0 = bt * 128
        col0 = dt * 8
        for ds in range(8):
          colv = jnp.full((16,), col0 + ds, jnp.int32)
          for l in range(8):
            rowv = (row0 + l * 16) + lanes
            v = plsc.load_gather(rows[b], [rowv, colv])
            tbuf[b][dt, bt, ds, pl.ds(l * 16, 16)] = v
        return carry
      lax.fori_loop(0, (d // 8) * n_bt, body, 0)

    for b in range(_NBUF):
      fire_gather(b, b)

    def group_body(g, carry):
      for b in range(_NBUF):
        i = g * _NBUF + b
        wait_gather(b)
        transpose_chunk(b)
        fire_gather(i + _NBUF, b)
        fire_out(i, b)
      for b in range(_NBUF):
        wait_out(b)
      return carry

    lax.fori_loop(0, n_groups - 1, group_body, 0)

    for b in range(_NBUF):
      i = (n_groups - 1) * _NBUF + b
      wait_gather(b)
      transpose_chunk(b)
      fire_out(i, b)
    for b in range(_NBUF):
      wait_out(b)

  return gather_kernel


def kernel(input, table):
  b, s, _ = input.shape
  vocab, d = table.shape
  n = b * s
  grid = -(-vocab // _RETILE_W)
  table_rows = jnp.reshape(_make_retile(vocab, d)(table.T),
                           (grid * _RETILE_W, d))
  v = jnp.reshape(jnp.transpose(input, (1, 2, 0)), (n,)).astype(jnp.int32)
  q = v & (_RETILE_W - 1)
  gidx = ((v >> 12) << 12) + ((q & (_RETILE_W // 2 - 1)) << 1) + (q >> 11)
  out5 = _make_gather(n, grid * _RETILE_W, d, s, b)(gidx, table_rows)
  return jnp.reshape(jnp.transpose(out5, (2, 4, 0, 1, 3)), (b, s, d))


# speedup vs baseline: 2.4065x; 2.3325x over previous
import functools

import jax
import jax.numpy as jnp
from jax import lax
from jax.experimental import pallas as pl
from jax.experimental.pallas import tpu as pltpu
from jax.experimental.pallas import tpu_sc as plsc

_NUM_CORES = 2
_NUM_SUBCORES = 16
_NUM_WORKERS = _NUM_CORES * _NUM_SUBCORES

_CHUNK = 512
_NBUF = 2

_RETILE_W = 4096


@functools.lru_cache(maxsize=None)
def _make_retile(vocab: int, d: int):
  grid = -(-vocab // _RETILE_W)

  def retile_body(tT_ref, out_ref):
    x = tT_ref[...]
    xt = jnp.transpose(x)
    y = jnp.concatenate([xt[:_RETILE_W // 2], xt[_RETILE_W // 2:]], axis=1)
    out_ref[...] = jnp.reshape(y, (_RETILE_W * d,))

  return pl.pallas_call(
      retile_body,
      grid=(grid,),
      in_specs=[pl.BlockSpec((d, _RETILE_W), lambda i: (0, i))],
      out_specs=pl.BlockSpec((_RETILE_W * d,), lambda i: (i,)),
      out_shape=jax.ShapeDtypeStruct((grid * _RETILE_W * d,), jnp.float32),
  )


@functools.lru_cache(maxsize=None)
def _make_gather(n: int, vocab2: int, d: int):
  n_per_w = n // _NUM_WORKERS
  n_chunks = n_per_w // _CHUNK
  n_groups = n_chunks // _NBUF
  assert n == n_per_w * _NUM_WORKERS
  assert n_per_w == n_chunks * _CHUNK
  assert n_chunks == n_groups * _NBUF
  mesh = plsc.VectorSubcoreMesh(
      core_axis_name="c", subcore_axis_name="s",
      num_cores=_NUM_CORES, num_subcores=_NUM_SUBCORES)

  @functools.partial(
      pl.kernel,
      out_type=jax.ShapeDtypeStruct((n, 2 * d), jnp.float32),
      mesh=mesh,
      compiler_params=pltpu.CompilerParams(use_tc_tiling_on_sc=False),
      scratch_types=[
          pltpu.VMEM((n_per_w,), jnp.int32),
          [pltpu.VMEM((_CHUNK, d), jnp.float32) for _ in range(_NBUF)],
          [pltpu.SemaphoreType.DMA for _ in range(_NBUF)],
          [pltpu.SemaphoreType.DMA for _ in range(_NBUF)],
      ],
  )
  def gather_kernel(idx_hbm, table_hbm, out_hbm, idx_all, rows, gsem, osem):
    wid = lax.axis_index("s") * _NUM_CORES + lax.axis_index("c")
    base = wid * n_per_w
    pltpu.sync_copy(idx_hbm.at[pl.ds(base, n_per_w)], idx_all)

    def fire_gather(chunk, b):
      pltpu.async_copy(
          table_hbm.at[idx_all.at[pl.ds(chunk * _CHUNK, _CHUNK)]],
          rows[b], gsem[b])

    def wait_gather(b):
      pltpu.make_async_copy(table_hbm.at[idx_all.at[pl.ds(0, _CHUNK)]],
                            rows[b], gsem[b]).wait()

    def fire_out(chunk, b):
      pltpu.async_copy(
          rows[b],
          out_hbm.at[pl.ds(base + chunk * _CHUNK, _CHUNK), pl.ds(0, d)],
          osem[b])

    def wait_out(b):
      pltpu.make_async_copy(rows[b],
                            out_hbm.at[pl.ds(base, _CHUNK), pl.ds(0, d)],
                            osem[b]).wait()

    for b in range(_NBUF):
      fire_gather(b, b)

    def group_body(g, carry):
      for b in range(_NBUF):
        wait_gather(b)
        fire_out(g * _NBUF + b, b)
      for b in range(_NBUF):
        wait_out(b)
        fire_gather((g + 1) * _NBUF + b, b)
      return carry

    lax.fori_loop(0, n_groups - 1, group_body, 0)

    for b in range(_NBUF):
      wait_gather(b)
      fire_out((n_groups - 1) * _NBUF + b, b)
    for b in range(_NBUF):
      wait_out(b)

  return gather_kernel


def kernel(input, table):
  b, s, _ = input.shape
  vocab, d = table.shape
  n = b * s
  grid = -(-vocab // _RETILE_W)
  table_rows = jnp.reshape(_make_retile(vocab, d)(table.T),
                           (grid * _RETILE_W, d))
  v = jnp.reshape(input, (n,)).astype(jnp.int32)
  q = v & (_RETILE_W - 1)
  gidx = ((v >> 12) << 12) + ((q & (_RETILE_W // 2 - 1)) << 1) + (q >> 11)
  out2 = _make_gather(n, grid * _RETILE_W, d)(gidx, table_rows)
  return jnp.reshape(out2[:, :d], (b, s, d))
